# Initial kernel scaffold; baseline (speedup 1.0000x reference)
import functools

import jax
import jax.numpy as jnp
from jax import lax
from jax.experimental import pallas as pl
from jax.experimental.pallas import tpu as pltpu
from jax.experimental.pallas import tpu_sc as plsc

N = 10000
NPAD = 10240
E = 320000
NC, NS = 2, 16
NW = NC * NS
CHUNKS = 80
EPT = CHUNKS * 128
TRASH = 10200
D1 = 16
D2P = 48
RB = 1024


def _make_deg_kernel():
    mesh = plsc.VectorSubcoreMesh(core_axis_name="c", subcore_axis_name="s")
    nrows = NPAD // 16

    @functools.partial(
        pl.kernel, mesh=mesh,
        out_type=jax.ShapeDtypeStruct((NC, nrows, 16), jnp.float32),
        scratch_types=[
            pltpu.VMEM((CHUNKS, 128), jnp.int32),
            pltpu.VMEM((nrows, 16), jnp.float32),
            pltpu.VMEM((5, 128), jnp.int32),
            pltpu.VMEM_SHARED((nrows, 16), jnp.float32),
        ],
    )
    def k(dst_hbm, out_hbm, idx_v, hist, rowid, acc):
        cid = lax.axis_index("c")
        sid = lax.axis_index("s")
        wid = sid * NC + cid

        def zrow(i, _):
            hist[i, pl.ds(0, 16)] = jnp.zeros((16,), jnp.float32)
            return 0
        lax.fori_loop(0, nrows, zrow, 0)
        for j in range(5):
            for c in range(8):
                rowid[j, pl.ds(c * 16, 16)] = (
                    lax.iota(jnp.int32, 16) + (j * 128 + c * 16))
        r0 = sid * (nrows // NS)
        pltpu.sync_copy(hist.at[pl.ds(r0, nrows // NS), :],
                        acc.at[pl.ds(r0, nrows // NS), :])
        pltpu.sync_copy(dst_hbm.at[wid], idx_v)
        plsc.subcore_barrier()

        ones = jnp.ones((16,), jnp.float32)

        def body(e, _):
            v = idx_v[e >> 3, pl.ds((e & 7) * 16, 16)]
            plsc.addupdate_scatter(hist, [v >> 4, v & 15], ones)
            return 0
        lax.fori_loop(0, EPT // 16, body, 0)

        for j in range(5):
            pltpu.sync_copy(hist.at[pl.ds(j * 128, 128), :],
                            acc.at[rowid.at[j]], add=True)
        plsc.subcore_barrier()
        pltpu.sync_copy(acc.at[pl.ds(r0, nrows // NS), :],
                        out_hbm.at[cid, pl.ds(r0, nrows // NS), :])

    return k


def _make_msg_kernel(D):
    mesh = plsc.VectorSubcoreMesh(core_axis_name="c", subcore_axis_name="s")
    rpt = NPAD // NS

    @functools.partial(
        pl.kernel, mesh=mesh,
        out_type=jax.ShapeDtypeStruct((NC, NPAD, D), jnp.float32),
        scratch_types=[
            pltpu.VMEM((CHUNKS, 128), jnp.int32),
            pltpu.VMEM((CHUNKS, 128), jnp.int32),
            pltpu.VMEM((128, D), jnp.float32),
            pltpu.VMEM_SHARED((NPAD, D), jnp.float32),
            pltpu.SemaphoreType.DMA,
        ],
    )
    def k(table_hbm, src_hbm, dst_hbm, out_hbm, src_v, dst_v, gbuf, acc, sem):
        cid = lax.axis_index("c")
        sid = lax.axis_index("s")
        wid = sid * NC + cid

        def zrow(i, _):
            for c in range(D // 16):
                gbuf[i, pl.ds(c * 16, 16)] = jnp.zeros((16,), jnp.float32)
            return 0
        lax.fori_loop(0, 128, zrow, 0)
        r0 = sid * rpt
        for b in range(rpt // 128):
            pltpu.sync_copy(gbuf, acc.at[pl.ds(r0 + b * 128, 128), :])
        pltpu.sync_copy(src_hbm.at[wid], src_v)
        pltpu.sync_copy(dst_hbm.at[wid], dst_v)
        plsc.subcore_barrier()

        def body(j, _):
            pltpu.async_copy(table_hbm.at[src_v.at[j]], gbuf, sem).wait()
            pltpu.sync_copy(gbuf, acc.at[dst_v.at[j]], add=True)
            return 0
        lax.fori_loop(0, CHUNKS, body, 0)

        plsc.subcore_barrier()
        pltpu.sync_copy(acc.at[pl.ds(r0, rpt), :],
                        out_hbm.at[cid, pl.ds(r0, rpt), :])

    return k


_deg_kernel = _make_deg_kernel()
_msg16 = _make_msg_kernel(D1)
_msg48 = _make_msg_kernel(D2P)


def _mm1_body(x_ref, w_ref, degp_ref, hs_ref, dinv_ref):
    deg = degp_ref[:, 0:1] + degp_ref[:, 1:2] + 1.0
    dinv = lax.rsqrt(deg)
    dinv_ref[...] = dinv
    h = jnp.dot(x_ref[...], w_ref[...], preferred_element_type=jnp.float32)
    hs_ref[...] = h * dinv


def _mm1(xp, W1, degp_t):
    return pl.pallas_call(
        _mm1_body,
        grid=(NPAD // RB,),
        in_specs=[
            pl.BlockSpec((RB, 128), lambda i: (i, 0)),
            pl.BlockSpec((128, D1), lambda i: (0, 0)),
            pl.BlockSpec((RB, 2), lambda i: (i, 0)),
        ],
        out_specs=[
            pl.BlockSpec((RB, D1), lambda i: (i, 0)),
            pl.BlockSpec((RB, 1), lambda i: (i, 0)),
        ],
        out_shape=[
            jax.ShapeDtypeStruct((NPAD, D1), jnp.float32),
            jax.ShapeDtypeStruct((NPAD, 1), jnp.float32),
        ],
    )(xp, W1, degp_t)


def _comb1_body(a0_ref, a1_ref, hs_ref, dinv_ref, b1_ref, w2_ref, hs2_ref):
    dinv = dinv_ref[...]
    s = (a0_ref[...] + a1_ref[...] + hs_ref[...]) * dinv + b1_ref[...]
    o1 = jnp.maximum(s, 0.0)
    hs2_ref[...] = jnp.dot(
        o1, w2_ref[...], preferred_element_type=jnp.float32) * dinv


def _comb1(a0, a1, hs1, dinv, b1r, W2p):
    return pl.pallas_call(
        _comb1_body,
        grid=(NPAD // RB,),
        in_specs=[
            pl.BlockSpec((RB, D1), lambda i: (i, 0)),
            pl.BlockSpec((RB, D1), lambda i: (i, 0)),
            pl.BlockSpec((RB, D1), lambda i: (i, 0)),
            pl.BlockSpec((RB, 1), lambda i: (i, 0)),
            pl.BlockSpec((1, D1), lambda i: (0, 0)),
            pl.BlockSpec((D1, D2P), lambda i: (0, 0)),
        ],
        out_specs=pl.BlockSpec((RB, D2P), lambda i: (i, 0)),
        out_shape=jax.ShapeDtypeStruct((NPAD, D2P), jnp.float32),
    )(a0, a1, hs1, dinv, b1r, W2p)


def _final_body(a0_ref, a1_ref, hs_ref, dinv_ref, b2_ref, out_ref):
    z = (a0_ref[...] + a1_ref[...] + hs_ref[...]) * dinv_ref[...] + b2_ref[...]
    mask = lax.broadcasted_iota(jnp.int32, z.shape, 1) < 40
    zm = jnp.where(mask, z, -jnp.inf)
    m = jnp.max(zm, axis=1, keepdims=True)
    e = jnp.where(mask, jnp.exp(z - m), 0.0)
    lse = jnp.log(jnp.sum(e, axis=1, keepdims=True))
    out_ref[...] = z - m - lse


def _final(a0, a1, hs2, dinv, b2r):
    return pl.pallas_call(
        _final_body,
        grid=(NPAD // RB,),
        in_specs=[
            pl.BlockSpec((RB, D2P), lambda i: (i, 0)),
            pl.BlockSpec((RB, D2P), lambda i: (i, 0)),
            pl.BlockSpec((RB, D2P), lambda i: (i, 0)),
            pl.BlockSpec((RB, 1), lambda i: (i, 0)),
            pl.BlockSpec((1, D2P), lambda i: (0, 0)),
        ],
        out_specs=pl.BlockSpec((RB, D2P), lambda i: (i, 0)),
        out_shape=jax.ShapeDtypeStruct((NPAD, D2P), jnp.float32),
    )(a0, a1, hs2, dinv, b2r)


def kernel(x, edge_index, W1, b1, W2, b2):
    src = edge_index[0].astype(jnp.int32)
    dst = edge_index[1].astype(jnp.int32)
    pad = NW * EPT - E
    srcp = jnp.concatenate(
        [src, jnp.zeros((pad,), jnp.int32)]).reshape(NW, CHUNKS, 128)
    dstp = jnp.concatenate(
        [dst, jnp.full((pad,), TRASH, jnp.int32)]).reshape(NW, CHUNKS, 128)
    xp = jnp.pad(x, ((0, NPAD - N), (0, 0)))
    W2p = jnp.pad(W2, ((0, 0), (0, D2P - W2.shape[1])))
    b1r = b1.reshape(1, D1)
    b2r = jnp.pad(b2, (0, D2P - b2.shape[0])).reshape(1, D2P)

    degp = _deg_kernel(dstp).reshape(NC, NPAD)
    degp_t = degp.T
    hs1, dinv = _mm1(xp, W1, degp_t)
    acc1 = _msg16(hs1, srcp, dstp)
    hs2 = _comb1(acc1[0], acc1[1], hs1, dinv, b1r, W2p)
    acc2 = _msg48(hs2, srcp, dstp)
    outp = _final(acc2[0], acc2[1], hs2, dinv, b2r)
    return outp[:N, :40]


# speedup vs baseline: 18.4774x; 18.4774x over previous
import functools

import jax
import jax.numpy as jnp
from jax import lax
from jax.experimental import pallas as pl
from jax.experimental.pallas import tpu as pltpu
from jax.experimental.pallas import tpu_sc as plsc

N = 10000
NPAD = 10240
E = 320000
NC, NS = 2, 16
NW = NC * NS
CHUNKS = 80
EPT = CHUNKS * 128
TRASH = 10200
D1 = 16
D2P = 48
RB = 1024


def _make_deg_kernel():
    mesh = plsc.VectorSubcoreMesh(core_axis_name="c", subcore_axis_name="s")
    rpt = NPAD // NS

    @functools.partial(
        pl.kernel, mesh=mesh,
        out_type=jax.ShapeDtypeStruct((NC, NPAD, 16), jnp.float32),
        compiler_params=pltpu.CompilerParams(use_tc_tiling_on_sc=False),
        scratch_types=[
            pltpu.VMEM((CHUNKS, 128), jnp.int32),
            pltpu.VMEM((128, 16), jnp.float32),
            pltpu.VMEM_SHARED((NPAD, 16), jnp.float32),
        ],
    )
    def k(dst_hbm, out_hbm, dst_v, obuf, acc):
        cid = lax.axis_index("c")
        sid = lax.axis_index("s")
        wid = sid * NC + cid

        def zrow(i, _):
            obuf[i, pl.ds(0, 16)] = jnp.zeros((16,), jnp.float32)
            return 0
        lax.fori_loop(0, 128, zrow, 0)
        r0 = sid * rpt
        for b in range(rpt // 128):
            pltpu.sync_copy(obuf, acc.at[pl.ds(r0 + b * 128, 128), :])

        def orow(i, _):
            obuf[i, pl.ds(0, 16)] = jnp.ones((16,), jnp.float32)
            return 0
        lax.fori_loop(0, 128, orow, 0)
        pltpu.sync_copy(dst_hbm.at[wid], dst_v)
        plsc.subcore_barrier()

        def body(j, _):
            pltpu.sync_copy(obuf, acc.at[dst_v.at[j]], add=True)
            return 0
        lax.fori_loop(0, CHUNKS, body, 0)

        plsc.subcore_barrier()
        pltpu.sync_copy(acc.at[pl.ds(r0, rpt), :],
                        out_hbm.at[cid, pl.ds(r0, rpt), :])

    return k


def _make_msg_kernel(D):
    mesh = plsc.VectorSubcoreMesh(core_axis_name="c", subcore_axis_name="s")
    rpt = NPAD // NS

    @functools.partial(
        pl.kernel, mesh=mesh,
        out_type=jax.ShapeDtypeStruct((NC, NPAD, D), jnp.float32),
        compiler_params=pltpu.CompilerParams(use_tc_tiling_on_sc=False),
        scratch_types=[
            pltpu.VMEM((CHUNKS, 128), jnp.int32),
            pltpu.VMEM((CHUNKS, 128), jnp.int32),
            pltpu.VMEM((128, D), jnp.float32),
            pltpu.VMEM_SHARED((NPAD, D), jnp.float32),
            pltpu.SemaphoreType.DMA,
        ],
    )
    def k(table_hbm, src_hbm, dst_hbm, out_hbm, src_v, dst_v, gbuf, acc, sem):
        cid = lax.axis_index("c")
        sid = lax.axis_index("s")
        wid = sid * NC + cid

        def zrow(i, _):
            for c in range(D // 16):
                gbuf[i, pl.ds(c * 16, 16)] = jnp.zeros((16,), jnp.float32)
            return 0
        lax.fori_loop(0, 128, zrow, 0)
        r0 = sid * rpt
        for b in range(rpt // 128):
            pltpu.sync_copy(gbuf, acc.at[pl.ds(r0 + b * 128, 128), :])
        pltpu.sync_copy(src_hbm.at[wid], src_v)
        pltpu.sync_copy(dst_hbm.at[wid], dst_v)
        plsc.subcore_barrier()

        def body(j, _):
            pltpu.async_copy(table_hbm.at[src_v.at[j]], gbuf, sem).wait()
            pltpu.sync_copy(gbuf, acc.at[dst_v.at[j]], add=True)
            return 0
        lax.fori_loop(0, CHUNKS, body, 0)

        plsc.subcore_barrier()
        pltpu.sync_copy(acc.at[pl.ds(r0, rpt), :],
                        out_hbm.at[cid, pl.ds(r0, rpt), :])

    return k


_deg_kernel = _make_deg_kernel()
_msg16 = _make_msg_kernel(D1)
_msg48 = _make_msg_kernel(D2P)


def _mm1_body(x_ref, w_ref, dp0_ref, dp1_ref, hs_ref, dinv_ref):
    deg = dp0_ref[:, 0:1] + dp1_ref[:, 0:1] + 1.0
    dinv = lax.rsqrt(deg)
    dinv_ref[...] = dinv
    h = jnp.dot(x_ref[...], w_ref[...], preferred_element_type=jnp.float32)
    hs_ref[...] = h * dinv


def _mm1(xp, W1, dp0, dp1):
    return pl.pallas_call(
        _mm1_body,
        grid=(NPAD // RB,),
        in_specs=[
            pl.BlockSpec((RB, 128), lambda i: (i, 0)),
            pl.BlockSpec((128, D1), lambda i: (0, 0)),
            pl.BlockSpec((RB, 16), lambda i: (i, 0)),
            pl.BlockSpec((RB, 16), lambda i: (i, 0)),
        ],
        out_specs=[
            pl.BlockSpec((RB, D1), lambda i: (i, 0)),
            pl.BlockSpec((RB, 1), lambda i: (i, 0)),
        ],
        out_shape=[
            jax.ShapeDtypeStruct((NPAD, D1), jnp.float32),
            jax.ShapeDtypeStruct((NPAD, 1), jnp.float32),
        ],
    )(xp, W1, dp0, dp1)


def _comb1_body(a0_ref, a1_ref, hs_ref, dinv_ref, b1_ref, w2_ref, hs2_ref):
    dinv = dinv_ref[...]
    s = (a0_ref[...] + a1_ref[...] + hs_ref[...]) * dinv + b1_ref[...]
    o1 = jnp.maximum(s, 0.0)
    hs2_ref[...] = jnp.dot(
        o1, w2_ref[...], preferred_element_type=jnp.float32) * dinv


def _comb1(a0, a1, hs1, dinv, b1r, W2p):
    return pl.pallas_call(
        _comb1_body,
        grid=(NPAD // RB,),
        in_specs=[
            pl.BlockSpec((RB, D1), lambda i: (i, 0)),
            pl.BlockSpec((RB, D1), lambda i: (i, 0)),
            pl.BlockSpec((RB, D1), lambda i: (i, 0)),
            pl.BlockSpec((RB, 1), lambda i: (i, 0)),
            pl.BlockSpec((1, D1), lambda i: (0, 0)),
            pl.BlockSpec((D1, D2P), lambda i: (0, 0)),
        ],
        out_specs=pl.BlockSpec((RB, D2P), lambda i: (i, 0)),
        out_shape=jax.ShapeDtypeStruct((NPAD, D2P), jnp.float32),
    )(a0, a1, hs1, dinv, b1r, W2p)


def _final_body(a0_ref, a1_ref, hs_ref, dinv_ref, b2_ref, out_ref):
    z = (a0_ref[...] + a1_ref[...] + hs_ref[...]) * dinv_ref[...] + b2_ref[...]
    mask = lax.broadcasted_iota(jnp.int32, z.shape, 1) < 40
    zm = jnp.where(mask, z, -jnp.inf)
    m = jnp.max(zm, axis=1, keepdims=True)
    e = jnp.where(mask, jnp.exp(z - m), 0.0)
    lse = jnp.log(jnp.sum(e, axis=1, keepdims=True))
    out_ref[...] = z - m - lse


def _final(a0, a1, hs2, dinv, b2r):
    return pl.pallas_call(
        _final_body,
        grid=(NPAD // RB,),
        in_specs=[
            pl.BlockSpec((RB, D2P), lambda i: (i, 0)),
            pl.BlockSpec((RB, D2P), lambda i: (i, 0)),
            pl.BlockSpec((RB, D2P), lambda i: (i, 0)),
            pl.BlockSpec((RB, 1), lambda i: (i, 0)),
            pl.BlockSpec((1, D2P), lambda i: (0, 0)),
        ],
        out_specs=pl.BlockSpec((RB, D2P), lambda i: (i, 0)),
        out_shape=jax.ShapeDtypeStruct((NPAD, D2P), jnp.float32),
    )(a0, a1, hs2, dinv, b2r)


def kernel(x, edge_index, W1, b1, W2, b2):
    src = edge_index[0].astype(jnp.int32)
    dst = edge_index[1].astype(jnp.int32)
    pad = NW * EPT - E
    srcp = jnp.concatenate(
        [src, jnp.zeros((pad,), jnp.int32)]).reshape(NW, CHUNKS, 128)
    dstp = jnp.concatenate(
        [dst, jnp.full((pad,), TRASH, jnp.int32)]).reshape(NW, CHUNKS, 128)
    xp = jnp.pad(x, ((0, NPAD - N), (0, 0)))
    W2p = jnp.pad(W2, ((0, 0), (0, D2P - W2.shape[1])))
    b1r = b1.reshape(1, D1)
    b2r = jnp.pad(b2, (0, D2P - b2.shape[0])).reshape(1, D2P)

    degp = _deg_kernel(dstp)
    hs1, dinv = _mm1(xp, W1, degp[0], degp[1])
    acc1 = _msg16(hs1, srcp, dstp)
    hs2 = _comb1(acc1[0], acc1[1], hs1, dinv, b1r, W2p)
    acc2 = _msg48(hs2, srcp, dstp)
    outp = _final(acc2[0], acc2[1], hs2, dinv, b2r)
    return outp[:N, :40]


# speedup vs baseline: 21.8814x; 1.1842x over previous
import functools

import jax
import jax.numpy as jnp
from jax import lax
from jax.experimental import pallas as pl
from jax.experimental.pallas import tpu as pltpu
from jax.experimental.pallas import tpu_sc as plsc

N = 10000
NPAD = 10240
E = 320000
NC, NS = 2, 16
NW = NC * NS
CHUNKS = 80
EPT = CHUNKS * 128
TRASH = 10200
D1 = 16
D2P = 48
RB = 1024
NBUF = 4


def _make_deg_kernel():
    mesh = plsc.VectorSubcoreMesh(core_axis_name="c", subcore_axis_name="s")
    rpt = NPAD // NS

    @functools.partial(
        pl.kernel, mesh=mesh,
        out_type=jax.ShapeDtypeStruct((NC, NPAD, 16), jnp.float32),
        compiler_params=pltpu.CompilerParams(use_tc_tiling_on_sc=False),
        scratch_types=[
            pltpu.VMEM((CHUNKS, 128), jnp.int32),
            pltpu.VMEM((128, 16), jnp.float32),
            pltpu.VMEM_SHARED((NPAD, 16), jnp.float32),
        ],
    )
    def k(dst_hbm, out_hbm, dst_v, obuf, acc):
        cid = lax.axis_index("c")
        sid = lax.axis_index("s")
        wid = sid * NC + cid

        def zrow(i, _):
            obuf[i, pl.ds(0, 16)] = jnp.zeros((16,), jnp.float32)
            return 0
        lax.fori_loop(0, 128, zrow, 0)
        r0 = sid * rpt
        for b in range(rpt // 128):
            pltpu.sync_copy(obuf, acc.at[pl.ds(r0 + b * 128, 128), :])

        def orow(i, _):
            obuf[i, pl.ds(0, 16)] = jnp.ones((16,), jnp.float32)
            return 0
        lax.fori_loop(0, 128, orow, 0)
        pltpu.sync_copy(dst_hbm.at[wid], dst_v)
        plsc.subcore_barrier()

        def body(j, _):
            pltpu.sync_copy(obuf, acc.at[dst_v.at[j]], add=True)
            return 0
        lax.fori_loop(0, CHUNKS, body, 0)

        plsc.subcore_barrier()
        pltpu.sync_copy(acc.at[pl.ds(r0, rpt), :],
                        out_hbm.at[cid, pl.ds(r0, rpt), :])

    return k


def _make_msg_kernel(D):
    mesh = plsc.VectorSubcoreMesh(core_axis_name="c", subcore_axis_name="s")
    rpt = NPAD // NS

    @functools.partial(
        pl.kernel, mesh=mesh,
        out_type=jax.ShapeDtypeStruct((NC, NPAD, D), jnp.float32),
        compiler_params=pltpu.CompilerParams(use_tc_tiling_on_sc=False),
        scratch_types=[
            pltpu.VMEM((CHUNKS, 128), jnp.int32),
            pltpu.VMEM((CHUNKS, 128), jnp.int32),
            [pltpu.VMEM((128, D), jnp.float32) for _ in range(NBUF)],
            [pltpu.SemaphoreType.DMA for _ in range(NBUF)],
            pltpu.VMEM_SHARED((NPAD, D), jnp.float32),
        ],
    )
    def k(table_hbm, src_hbm, dst_hbm, out_hbm, src_v, dst_v, gbufs, sems, acc):
        cid = lax.axis_index("c")
        sid = lax.axis_index("s")
        wid = sid * NC + cid

        gbuf0 = gbufs[0]

        def zrow(i, _):
            for c in range(D // 16):
                gbuf0[i, pl.ds(c * 16, 16)] = jnp.zeros((16,), jnp.float32)
            return 0
        lax.fori_loop(0, 128, zrow, 0)
        r0 = sid * rpt
        for b in range(rpt // 128):
            pltpu.sync_copy(gbuf0, acc.at[pl.ds(r0 + b * 128, 128), :])
        pltpu.sync_copy(src_hbm.at[wid], src_v)
        pltpu.sync_copy(dst_hbm.at[wid], dst_v)
        plsc.subcore_barrier()

        for b in range(NBUF):
            pltpu.async_copy(table_hbm.at[src_v.at[b]], gbufs[b], sems[b])

        def group(g, _):
            base = g * NBUF
            for b in range(NBUF):
                j = base + b
                pltpu.make_async_copy(
                    table_hbm.at[src_v.at[j]], gbufs[b], sems[b]).wait()
                pltpu.sync_copy(gbufs[b], acc.at[dst_v.at[j]], add=True)
                jn = j + NBUF

                @pl.when(jn < CHUNKS)
                def _():
                    pltpu.async_copy(
                        table_hbm.at[src_v.at[jn]], gbufs[b], sems[b])
            return 0
        lax.fori_loop(0, CHUNKS // NBUF, group, 0)

        plsc.subcore_barrier()
        pltpu.sync_copy(acc.at[pl.ds(r0, rpt), :],
                        out_hbm.at[cid, pl.ds(r0, rpt), :])

    return k


_deg_kernel = _make_deg_kernel()
_msg16 = _make_msg_kernel(D1)
_msg48 = _make_msg_kernel(D2P)


def _mm1_body(x_ref, w_ref, dp0_ref, dp1_ref, hs_ref, dinv_ref):
    deg = dp0_ref[:, 0:1] + dp1_ref[:, 0:1] + 1.0
    dinv = lax.rsqrt(deg)
    dinv_ref[...] = dinv
    h = jnp.dot(x_ref[...], w_ref[...], preferred_element_type=jnp.float32)
    hs_ref[...] = h * dinv


def _mm1(xp, W1, dp0, dp1):
    return pl.pallas_call(
        _mm1_body,
        grid=(NPAD // RB,),
        in_specs=[
            pl.BlockSpec((RB, 128), lambda i: (i, 0)),
            pl.BlockSpec((128, D1), lambda i: (0, 0)),
            pl.BlockSpec((RB, 16), lambda i: (i, 0)),
            pl.BlockSpec((RB, 16), lambda i: (i, 0)),
        ],
        out_specs=[
            pl.BlockSpec((RB, D1), lambda i: (i, 0)),
            pl.BlockSpec((RB, 1), lambda i: (i, 0)),
        ],
        out_shape=[
            jax.ShapeDtypeStruct((NPAD, D1), jnp.float32),
            jax.ShapeDtypeStruct((NPAD, 1), jnp.float32),
        ],
    )(xp, W1, dp0, dp1)


def _comb1_body(a0_ref, a1_ref, hs_ref, dinv_ref, b1_ref, w2_ref, hs2_ref):
    dinv = dinv_ref[...]
    s = (a0_ref[...] + a1_ref[...] + hs_ref[...]) * dinv + b1_ref[...]
    o1 = jnp.maximum(s, 0.0)
    hs2_ref[...] = jnp.dot(
        o1, w2_ref[...], preferred_element_type=jnp.float32) * dinv


def _comb1(a0, a1, hs1, dinv, b1r, W2p):
    return pl.pallas_call(
        _comb1_body,
        grid=(NPAD // RB,),
        in_specs=[
            pl.BlockSpec((RB, D1), lambda i: (i, 0)),
            pl.BlockSpec((RB, D1), lambda i: (i, 0)),
            pl.BlockSpec((RB, D1), lambda i: (i, 0)),
            pl.BlockSpec((RB, 1), lambda i: (i, 0)),
            pl.BlockSpec((1, D1), lambda i: (0, 0)),
            pl.BlockSpec((D1, D2P), lambda i: (0, 0)),
        ],
        out_specs=pl.BlockSpec((RB, D2P), lambda i: (i, 0)),
        out_shape=jax.ShapeDtypeStruct((NPAD, D2P), jnp.float32),
    )(a0, a1, hs1, dinv, b1r, W2p)


def _final_body(a0_ref, a1_ref, hs_ref, dinv_ref, b2_ref, out_ref):
    z = (a0_ref[...] + a1_ref[...] + hs_ref[...]) * dinv_ref[...] + b2_ref[...]
    mask = lax.broadcasted_iota(jnp.int32, z.shape, 1) < 40
    zm = jnp.where(mask, z, -jnp.inf)
    m = jnp.max(zm, axis=1, keepdims=True)
    e = jnp.where(mask, jnp.exp(z - m), 0.0)
    lse = jnp.log(jnp.sum(e, axis=1, keepdims=True))
    out_ref[...] = z - m - lse


def _final(a0, a1, hs2, dinv, b2r):
    return pl.pallas_call(
        _final_body,
        grid=(NPAD // RB,),
        in_specs=[
            pl.BlockSpec((RB, D2P), lambda i: (i, 0)),
            pl.BlockSpec((RB, D2P), lambda i: (i, 0)),
            pl.BlockSpec((RB, D2P), lambda i: (i, 0)),
            pl.BlockSpec((RB, 1), lambda i: (i, 0)),
            pl.BlockSpec((1, D2P), lambda i: (0, 0)),
        ],
        out_specs=pl.BlockSpec((RB, D2P), lambda i: (i, 0)),
        out_shape=jax.ShapeDtypeStruct((NPAD, D2P), jnp.float32),
    )(a0, a1, hs2, dinv, b2r)


def kernel(x, edge_index, W1, b1, W2, b2):
    src = edge_index[0].astype(jnp.int32)
    dst = edge_index[1].astype(jnp.int32)
    pad = NW * EPT - E
    srcp = jnp.concatenate(
        [src, jnp.zeros((pad,), jnp.int32)]).reshape(NW, CHUNKS, 128)
    dstp = jnp.concatenate(
        [dst, jnp.full((pad,), TRASH, jnp.int32)]).reshape(NW, CHUNKS, 128)
    xp = jnp.pad(x, ((0, NPAD - N), (0, 0)))
    W2p = jnp.pad(W2, ((0, 0), (0, D2P - W2.shape[1])))
    b1r = b1.reshape(1, D1)
    b2r = jnp.pad(b2, (0, D2P - b2.shape[0])).reshape(1, D2P)

    degp = _deg_kernel(dstp)
    hs1, dinv = _mm1(xp, W1, degp[0], degp[1])
    acc1 = _msg16(hs1, srcp, dstp)
    hs2 = _comb1(acc1[0], acc1[1], hs1, dinv, b1r, W2p)
    acc2 = _msg48(hs2, srcp, dstp)
    outp = _final(acc2[0], acc2[1], hs2, dinv, b2r)
    return outp[:N, :40]


# speedup vs baseline: 41.7757x; 1.9092x over previous
import functools

import jax
import jax.numpy as jnp
from jax import lax
from jax.experimental import pallas as pl
from jax.experimental.pallas import tpu as pltpu
from jax.experimental.pallas import tpu_sc as plsc

N = 10000
NPAD = 10240
E = 320000
NC, NS = 2, 16
NW = NC * NS
CHUNKS = 80
EPT = CHUNKS * 128
TRASH = 10200
D1 = 16
D2P = 40
RB = 1024
NBUF = 4


def _make_deg_kernel():
    mesh = plsc.VectorSubcoreMesh(core_axis_name="c", subcore_axis_name="s")
    rpt = NPAD // NS

    @functools.partial(
        pl.kernel, mesh=mesh,
        out_type=jax.ShapeDtypeStruct((NC, NPAD, 16), jnp.float32),
        compiler_params=pltpu.CompilerParams(use_tc_tiling_on_sc=False),
        scratch_types=[
            pltpu.VMEM((CHUNKS, 128), jnp.int32),
            pltpu.VMEM((128, 16), jnp.float32),
            pltpu.VMEM_SHARED((NPAD, 16), jnp.float32),
        ],
    )
    def k(dst_hbm, out_hbm, dst_v, obuf, acc):
        cid = lax.axis_index("c")
        sid = lax.axis_index("s")
        wid = sid * NC + cid

        def zrow(i, _):
            obuf[i, pl.ds(0, 16)] = jnp.zeros((16,), jnp.float32)
            return 0
        lax.fori_loop(0, 128, zrow, 0)
        r0 = sid * rpt
        for b in range(rpt // 128):
            pltpu.sync_copy(obuf, acc.at[pl.ds(r0 + b * 128, 128), :])

        def orow(i, _):
            obuf[i, pl.ds(0, 16)] = jnp.ones((16,), jnp.float32)
            return 0
        lax.fori_loop(0, 128, orow, 0)
        pltpu.sync_copy(dst_hbm.at[wid], dst_v)
        plsc.subcore_barrier()

        def body(j, _):
            pltpu.sync_copy(obuf, acc.at[dst_v.at[j]], add=True)
            return 0
        lax.fori_loop(0, CHUNKS, body, 0)

        plsc.subcore_barrier()
        pltpu.sync_copy(acc.at[pl.ds(r0, rpt), :],
                        out_hbm.at[cid, pl.ds(r0, rpt), :])

    return k


def _make_msg_kernel(D, stage):
    mesh = plsc.VectorSubcoreMesh(core_axis_name="c", subcore_axis_name="s")
    rpt = NPAD // NS

    scratch = [
        pltpu.VMEM((CHUNKS, 128), jnp.int32),
        pltpu.VMEM((CHUNKS, 128), jnp.int32),
        [pltpu.VMEM((128, D), jnp.float32) for _ in range(NBUF)],
        [pltpu.SemaphoreType.DMA for _ in range(NBUF)],
        pltpu.VMEM_SHARED((NPAD, D), jnp.float32),
    ]
    if stage:
        scratch += [
            pltpu.VMEM((NPAD // NS, D), jnp.float32),
            pltpu.SemaphoreType.DMA,
            pltpu.VMEM_SHARED((NPAD, D), jnp.float32),
        ]

    @functools.partial(
        pl.kernel, mesh=mesh,
        out_type=jax.ShapeDtypeStruct((NC, NPAD, D), jnp.float32),
        compiler_params=pltpu.CompilerParams(use_tc_tiling_on_sc=False),
        scratch_types=scratch,
    )
    def k(table_hbm, src_hbm, dst_hbm, zeros_hbm, out_hbm, src_v, dst_v,
          gbufs, sems, acc, *stage_refs):
        cid = lax.axis_index("c")
        sid = lax.axis_index("s")
        wid = sid * NC + cid
        r0 = sid * rpt

        if stage:
            stage_v, sem_s, table_sh = stage_refs
            pltpu.async_copy(table_hbm.at[pl.ds(r0, rpt), :], stage_v, sem_s)
        else:
            table_sh = table_hbm

        pltpu.sync_copy(zeros_hbm, acc.at[pl.ds(r0, rpt), :])
        pltpu.sync_copy(src_hbm.at[wid], src_v)
        pltpu.sync_copy(dst_hbm.at[wid], dst_v)
        if stage:
            pltpu.make_async_copy(
                table_hbm.at[pl.ds(r0, rpt), :], stage_v, sem_s).wait()
            pltpu.sync_copy(stage_v, table_sh.at[pl.ds(r0, rpt), :])
        plsc.subcore_barrier()

        for b in range(NBUF):
            pltpu.async_copy(table_sh.at[src_v.at[b]], gbufs[b], sems[b])

        def group(g, _):
            base = g * NBUF
            for b in range(NBUF):
                j = base + b
                pltpu.make_async_copy(
                    table_sh.at[src_v.at[j]], gbufs[b], sems[b]).wait()
                pltpu.sync_copy(gbufs[b], acc.at[dst_v.at[j]], add=True)
                jn = j + NBUF

                @pl.when(jn < CHUNKS)
                def _():
                    pltpu.async_copy(
                        table_sh.at[src_v.at[jn]], gbufs[b], sems[b])
            return 0
        lax.fori_loop(0, CHUNKS // NBUF, group, 0)

        plsc.subcore_barrier()
        pltpu.sync_copy(acc.at[pl.ds(r0, rpt), :],
                        out_hbm.at[cid, pl.ds(r0, rpt), :])

    return k


_deg_kernel = _make_deg_kernel()
_msg16 = _make_msg_kernel(D1, stage=True)
_msg48 = _make_msg_kernel(D2P, stage=True)


def _mm1_body(x_ref, w_ref, dp0_ref, dp1_ref, hs_ref, dinv_ref):
    deg = dp0_ref[...] + dp1_ref[...] + 1.0
    dinv = lax.rsqrt(deg)
    dinv_ref[...] = dinv
    h = jnp.dot(x_ref[...], w_ref[...], preferred_element_type=jnp.float32)
    hs_ref[...] = h * dinv


def _mm1(xp, W1, dp0, dp1):
    return pl.pallas_call(
        _mm1_body,
        grid=(NPAD // RB,),
        in_specs=[
            pl.BlockSpec((RB, 128), lambda i: (i, 0)),
            pl.BlockSpec((128, D1), lambda i: (0, 0)),
            pl.BlockSpec((RB, 1), lambda i: (i, 0)),
            pl.BlockSpec((RB, 1), lambda i: (i, 0)),
        ],
        out_specs=[
            pl.BlockSpec((RB, D1), lambda i: (i, 0)),
            pl.BlockSpec((RB, 1), lambda i: (i, 0)),
        ],
        out_shape=[
            jax.ShapeDtypeStruct((NPAD, D1), jnp.float32),
            jax.ShapeDtypeStruct((NPAD, 1), jnp.float32),
        ],
    )(xp, W1, dp0, dp1)


def _comb1_body(a0_ref, a1_ref, hs_ref, dinv_ref, b1_ref, w2_ref, hs2_ref):
    dinv = dinv_ref[...]
    s = (a0_ref[...] + a1_ref[...] + hs_ref[...]) * dinv + b1_ref[...]
    o1 = jnp.maximum(s, 0.0)
    hs2_ref[...] = jnp.dot(
        o1, w2_ref[...], preferred_element_type=jnp.float32) * dinv


def _comb1(a0, a1, hs1, dinv, b1r, W2p):
    return pl.pallas_call(
        _comb1_body,
        grid=(NPAD // RB,),
        in_specs=[
            pl.BlockSpec((RB, D1), lambda i: (i, 0)),
            pl.BlockSpec((RB, D1), lambda i: (i, 0)),
            pl.BlockSpec((RB, D1), lambda i: (i, 0)),
            pl.BlockSpec((RB, 1), lambda i: (i, 0)),
            pl.BlockSpec((1, D1), lambda i: (0, 0)),
            pl.BlockSpec((D1, D2P), lambda i: (0, 0)),
        ],
        out_specs=pl.BlockSpec((RB, D2P), lambda i: (i, 0)),
        out_shape=jax.ShapeDtypeStruct((NPAD, D2P), jnp.float32),
    )(a0, a1, hs1, dinv, b1r, W2p)


def _final_body(a0_ref, a1_ref, hs_ref, dinv_ref, b2_ref, out_ref):
    z = (a0_ref[...] + a1_ref[...] + hs_ref[...]) * dinv_ref[...] + b2_ref[...]
    m = jnp.max(z, axis=1, keepdims=True)
    lse = jnp.log(jnp.sum(jnp.exp(z - m), axis=1, keepdims=True))
    out_ref[...] = z - m - lse


def _final(a0, a1, hs2, dinv, b2r):
    return pl.pallas_call(
        _final_body,
        grid=(NPAD // RB,),
        in_specs=[
            pl.BlockSpec((RB, D2P), lambda i: (i, 0)),
            pl.BlockSpec((RB, D2P), lambda i: (i, 0)),
            pl.BlockSpec((RB, D2P), lambda i: (i, 0)),
            pl.BlockSpec((RB, 1), lambda i: (i, 0)),
            pl.BlockSpec((1, D2P), lambda i: (0, 0)),
        ],
        out_specs=pl.BlockSpec((RB, D2P), lambda i: (i, 0)),
        out_shape=jax.ShapeDtypeStruct((NPAD, D2P), jnp.float32),
    )(a0, a1, hs2, dinv, b2r)


def kernel(x, edge_index, W1, b1, W2, b2):
    src = edge_index[0].astype(jnp.int32)
    dst = edge_index[1].astype(jnp.int32)
    pad = NW * EPT - E
    srcp = jnp.concatenate(
        [src, jnp.zeros((pad,), jnp.int32)]).reshape(NW, CHUNKS, 128)
    dstp = jnp.concatenate(
        [dst, jnp.full((pad,), TRASH, jnp.int32)]).reshape(NW, CHUNKS, 128)
    xp = jnp.pad(x, ((0, NPAD - N), (0, 0)))
    b1r = b1.reshape(1, D1)
    b2r = b2.reshape(1, D2P)
    z16 = jnp.zeros((NPAD // NS, D1), jnp.float32)
    z40 = jnp.zeros((NPAD // NS, D2P), jnp.float32)

    degp = _deg_kernel(dstp)
    hs1, dinv = _mm1(xp, W1, degp[0, :, :1], degp[1, :, :1])
    acc1 = _msg16(hs1, srcp, dstp, z16)
    hs2 = _comb1(acc1[0], acc1[1], hs1, dinv, b1r, W2)
    acc2 = _msg48(hs2, srcp, dstp, z40)
    outp = _final(acc2[0], acc2[1], hs2, dinv, b2r)
    return outp[:N]


# speedup vs baseline: 49.9997x; 1.1969x over previous
import functools

import jax
import jax.numpy as jnp
from jax import lax
from jax.experimental import pallas as pl
from jax.experimental.pallas import tpu as pltpu
from jax.experimental.pallas import tpu_sc as plsc

N = 10000
E = 320000
NC, NS = 2, 16
NW = NC * NS
CHUNKS = 80
EC = 125
RPT = N // NS
D1 = 16
D2P = 40
RB = 2000
NBUF = 4


def _make_deg_kernel():
    mesh = plsc.VectorSubcoreMesh(core_axis_name="c", subcore_axis_name="s")

    @functools.partial(
        pl.kernel, mesh=mesh,
        out_type=jax.ShapeDtypeStruct((NC, N, 16), jnp.float32),
        compiler_params=pltpu.CompilerParams(use_tc_tiling_on_sc=False),
        scratch_types=[
            pltpu.VMEM((CHUNKS, EC), jnp.int32),
            pltpu.VMEM((EC, 16), jnp.float32),
            pltpu.VMEM_SHARED((N, 16), jnp.float32),
        ],
    )
    def k(dst_hbm, out_hbm, dst_v, obuf, acc):
        cid = lax.axis_index("c")
        sid = lax.axis_index("s")
        wid = sid * NC + cid

        def zrow(i, _):
            obuf[i, pl.ds(0, 16)] = jnp.zeros((16,), jnp.float32)
            return 0
        lax.fori_loop(0, EC, zrow, 0)
        r0 = sid * RPT
        for b in range(RPT // EC):
            pltpu.sync_copy(obuf, acc.at[pl.ds(r0 + b * EC, EC), :])

        def orow(i, _):
            obuf[i, pl.ds(0, 16)] = jnp.ones((16,), jnp.float32)
            return 0
        lax.fori_loop(0, EC, orow, 0)
        pltpu.sync_copy(dst_hbm.at[wid], dst_v)
        plsc.subcore_barrier()

        def body(j, _):
            pltpu.sync_copy(obuf, acc.at[dst_v.at[j]], add=True)
            return 0
        lax.fori_loop(0, CHUNKS, body, 0)

        plsc.subcore_barrier()
        pltpu.sync_copy(acc.at[pl.ds(r0, RPT), :],
                        out_hbm.at[cid, pl.ds(r0, RPT), :])

    return k


def _make_msg_kernel(D):
    mesh = plsc.VectorSubcoreMesh(core_axis_name="c", subcore_axis_name="s")

    @functools.partial(
        pl.kernel, mesh=mesh,
        out_type=jax.ShapeDtypeStruct((NC, N, D), jnp.float32),
        compiler_params=pltpu.CompilerParams(use_tc_tiling_on_sc=False),
        scratch_types=[
            pltpu.VMEM((CHUNKS, EC), jnp.int32),
            pltpu.VMEM((CHUNKS, EC), jnp.int32),
            [pltpu.VMEM((EC, D), jnp.float32) for _ in range(NBUF)],
            [pltpu.SemaphoreType.DMA for _ in range(NBUF)],
            pltpu.VMEM((RPT, D), jnp.float32),
            pltpu.SemaphoreType.DMA,
            pltpu.VMEM_SHARED((N, D), jnp.float32),
            pltpu.VMEM_SHARED((N, D), jnp.float32),
        ],
    )
    def k(table_hbm, src_hbm, dst_hbm, zeros_hbm, out_hbm, src_v, dst_v,
          gbufs, sems, stage_v, sem_s, table_sh, acc):
        cid = lax.axis_index("c")
        sid = lax.axis_index("s")
        wid = sid * NC + cid
        r0 = sid * RPT

        pltpu.async_copy(table_hbm.at[pl.ds(r0, RPT), :], stage_v, sem_s)
        pltpu.sync_copy(zeros_hbm, acc.at[pl.ds(r0, RPT), :])
        pltpu.sync_copy(src_hbm.at[wid], src_v)
        pltpu.sync_copy(dst_hbm.at[wid], dst_v)
        pltpu.make_async_copy(
            table_hbm.at[pl.ds(r0, RPT), :], stage_v, sem_s).wait()
        pltpu.sync_copy(stage_v, table_sh.at[pl.ds(r0, RPT), :])
        plsc.subcore_barrier()

        for b in range(NBUF):
            pltpu.async_copy(table_sh.at[src_v.at[b]], gbufs[b], sems[b])

        def group(g, _):
            base = g * NBUF
            for b in range(NBUF):
                j = base + b
                pltpu.make_async_copy(
                    table_sh.at[src_v.at[j]], gbufs[b], sems[b]).wait()
                pltpu.sync_copy(gbufs[b], acc.at[dst_v.at[j]], add=True)
                jn = j + NBUF

                @pl.when(jn < CHUNKS)
                def _():
                    pltpu.async_copy(
                        table_sh.at[src_v.at[jn]], gbufs[b], sems[b])
            return 0
        lax.fori_loop(0, CHUNKS // NBUF, group, 0)

        plsc.subcore_barrier()
        pltpu.sync_copy(acc.at[pl.ds(r0, RPT), :],
                        out_hbm.at[cid, pl.ds(r0, RPT), :])

    return k


_deg_kernel = _make_deg_kernel()
_msg16 = _make_msg_kernel(D1)
_msg48 = _make_msg_kernel(D2P)


def _mm1_body(x_ref, w_ref, dp0_ref, dp1_ref, hs_ref, dinv_ref):
    deg = dp0_ref[0, :, 0:1] + dp1_ref[0, :, 0:1] + 1.0
    dinv = lax.rsqrt(deg)
    dinv_ref[...] = dinv
    h = jnp.dot(x_ref[...], w_ref[...], preferred_element_type=jnp.float32)
    hs_ref[...] = h * dinv


def _mm1(x, W1, degp):
    return pl.pallas_call(
        _mm1_body,
        grid=(N // RB,),
        in_specs=[
            pl.BlockSpec((RB, 128), lambda i: (i, 0)),
            pl.BlockSpec((128, D1), lambda i: (0, 0)),
            pl.BlockSpec((1, RB, 16), lambda i: (0, i, 0)),
            pl.BlockSpec((1, RB, 16), lambda i: (1, i, 0)),
        ],
        out_specs=[
            pl.BlockSpec((RB, D1), lambda i: (i, 0)),
            pl.BlockSpec((RB, 1), lambda i: (i, 0)),
        ],
        out_shape=[
            jax.ShapeDtypeStruct((N, D1), jnp.float32),
            jax.ShapeDtypeStruct((N, 1), jnp.float32),
        ],
    )(x, W1, degp, degp)


def _comb1_body(a0_ref, a1_ref, hs_ref, dinv_ref, b1_ref, w2_ref, hs2_ref):
    dinv = dinv_ref[...]
    s = (a0_ref[0] + a1_ref[0] + hs_ref[...]) * dinv + b1_ref[...]
    o1 = jnp.maximum(s, 0.0)
    hs2_ref[...] = jnp.dot(
        o1, w2_ref[...], preferred_element_type=jnp.float32) * dinv


def _comb1(acc1, hs1, dinv, b1r, W2):
    return pl.pallas_call(
        _comb1_body,
        grid=(N // RB,),
        in_specs=[
            pl.BlockSpec((1, RB, D1), lambda i: (0, i, 0)),
            pl.BlockSpec((1, RB, D1), lambda i: (1, i, 0)),
            pl.BlockSpec((RB, D1), lambda i: (i, 0)),
            pl.BlockSpec((RB, 1), lambda i: (i, 0)),
            pl.BlockSpec((1, D1), lambda i: (0, 0)),
            pl.BlockSpec((D1, D2P), lambda i: (0, 0)),
        ],
        out_specs=pl.BlockSpec((RB, D2P), lambda i: (i, 0)),
        out_shape=jax.ShapeDtypeStruct((N, D2P), jnp.float32),
    )(acc1, acc1, hs1, dinv, b1r, W2)


def _final_body(a0_ref, a1_ref, hs_ref, dinv_ref, b2_ref, out_ref):
    z = (a0_ref[0] + a1_ref[0] + hs_ref[...]) * dinv_ref[...] + b2_ref[...]
    m = jnp.max(z, axis=1, keepdims=True)
    lse = jnp.log(jnp.sum(jnp.exp(z - m), axis=1, keepdims=True))
    out_ref[...] = z - m - lse


def _final(acc2, hs2, dinv, b2r):
    return pl.pallas_call(
        _final_body,
        grid=(N // RB,),
        in_specs=[
            pl.BlockSpec((1, RB, D2P), lambda i: (0, i, 0)),
            pl.BlockSpec((1, RB, D2P), lambda i: (1, i, 0)),
            pl.BlockSpec((RB, D2P), lambda i: (i, 0)),
            pl.BlockSpec((RB, 1), lambda i: (i, 0)),
            pl.BlockSpec((1, D2P), lambda i: (0, 0)),
        ],
        out_specs=pl.BlockSpec((RB, D2P), lambda i: (i, 0)),
        out_shape=jax.ShapeDtypeStruct((N, D2P), jnp.float32),
    )(acc2, acc2, hs2, dinv, b2r)


def kernel(x, edge_index, W1, b1, W2, b2):
    er = jnp.asarray(edge_index, jnp.int32).reshape(2, NW, CHUNKS, EC)
    srcp = er[0]
    dstp = er[1]
    b1r = b1.reshape(1, D1)
    b2r = b2.reshape(1, D2P)
    z16 = jnp.zeros((RPT, D1), jnp.float32)
    z40 = jnp.zeros((RPT, D2P), jnp.float32)

    degp = _deg_kernel(dstp)
    hs1, dinv = _mm1(x, W1, degp)
    acc1 = _msg16(hs1, srcp, dstp, z16)
    hs2 = _comb1(acc1, hs1, dinv, b1r, W2)
    acc2 = _msg48(hs2, srcp, dstp, z40)
    return _final(acc2, hs2, dinv, b2r)


# speedup vs baseline: 50.2486x; 1.0050x over previous
import functools

import jax
import jax.numpy as jnp
from jax import lax
from jax.experimental import pallas as pl
from jax.experimental.pallas import tpu as pltpu
from jax.experimental.pallas import tpu_sc as plsc

N = 10000
E = 320000
NC, NS = 2, 16
NW = NC * NS
CHUNKS = 80
EC = 125
RPT = N // NS
D1 = 16
D2P = 40
RB = 2000
NBUF = 4


def _make_deg_kernel():
    mesh = plsc.VectorSubcoreMesh(core_axis_name="c", subcore_axis_name="s")

    @functools.partial(
        pl.kernel, mesh=mesh,
        out_type=jax.ShapeDtypeStruct((NC, N, 16), jnp.float32),
        compiler_params=pltpu.CompilerParams(use_tc_tiling_on_sc=False),
        scratch_types=[
            pltpu.VMEM((CHUNKS, EC), jnp.int32),
            pltpu.VMEM((EC, 16), jnp.float32),
            pltpu.VMEM_SHARED((N, 16), jnp.float32),
        ],
    )
    def k(er_hbm, out_hbm, dst_v, obuf, acc):
        cid = lax.axis_index("c")
        sid = lax.axis_index("s")
        wid = sid * NC + cid

        def zrow(i, _):
            obuf[i, pl.ds(0, 16)] = jnp.zeros((16,), jnp.float32)
            return 0
        lax.fori_loop(0, EC, zrow, 0)
        r0 = sid * RPT
        for b in range(RPT // EC):
            pltpu.sync_copy(obuf, acc.at[pl.ds(r0 + b * EC, EC), :])

        def orow(i, _):
            obuf[i, pl.ds(0, 16)] = jnp.ones((16,), jnp.float32)
            return 0
        lax.fori_loop(0, EC, orow, 0)
        pltpu.sync_copy(er_hbm.at[1, wid], dst_v)
        plsc.subcore_barrier()

        def body(j, _):
            pltpu.sync_copy(obuf, acc.at[dst_v.at[j]], add=True)
            return 0
        lax.fori_loop(0, CHUNKS, body, 0)

        plsc.subcore_barrier()
        pltpu.sync_copy(acc.at[pl.ds(r0, RPT), :],
                        out_hbm.at[cid, pl.ds(r0, RPT), :])

    return k


def _make_msg_kernel(D):
    mesh = plsc.VectorSubcoreMesh(core_axis_name="c", subcore_axis_name="s")

    @functools.partial(
        pl.kernel, mesh=mesh,
        out_type=jax.ShapeDtypeStruct((NC, N, D), jnp.float32),
        compiler_params=pltpu.CompilerParams(use_tc_tiling_on_sc=False),
        scratch_types=[
            pltpu.VMEM((CHUNKS, EC), jnp.int32),
            pltpu.VMEM((CHUNKS, EC), jnp.int32),
            [pltpu.VMEM((EC, D), jnp.float32) for _ in range(NBUF)],
            [pltpu.SemaphoreType.DMA for _ in range(NBUF)],
            pltpu.VMEM((RPT, D), jnp.float32),
            pltpu.SemaphoreType.DMA,
            pltpu.VMEM_SHARED((N, D), jnp.float32),
            pltpu.VMEM_SHARED((N, D), jnp.float32),
        ],
    )
    def k(table_hbm, er_hbm, zeros_hbm, out_hbm, src_v, dst_v,
          gbufs, sems, stage_v, sem_s, table_sh, acc):
        cid = lax.axis_index("c")
        sid = lax.axis_index("s")
        wid = sid * NC + cid
        r0 = sid * RPT

        pltpu.async_copy(table_hbm.at[pl.ds(r0, RPT), :], stage_v, sem_s)
        pltpu.sync_copy(zeros_hbm, acc.at[pl.ds(r0, RPT), :])
        pltpu.sync_copy(er_hbm.at[0, wid], src_v)
        pltpu.sync_copy(er_hbm.at[1, wid], dst_v)
        pltpu.make_async_copy(
            table_hbm.at[pl.ds(r0, RPT), :], stage_v, sem_s).wait()
        pltpu.sync_copy(stage_v, table_sh.at[pl.ds(r0, RPT), :])
        plsc.subcore_barrier()

        for b in range(NBUF):
            pltpu.async_copy(table_sh.at[src_v.at[b]], gbufs[b], sems[b])

        def group(g, _):
            base = g * NBUF
            for b in range(NBUF):
                j = base + b
                pltpu.make_async_copy(
                    table_sh.at[src_v.at[j]], gbufs[b], sems[b]).wait()
                pltpu.sync_copy(gbufs[b], acc.at[dst_v.at[j]], add=True)
                jn = j + NBUF

                @pl.when(jn < CHUNKS)
                def _():
                    pltpu.async_copy(
                        table_sh.at[src_v.at[jn]], gbufs[b], sems[b])
            return 0
        lax.fori_loop(0, CHUNKS // NBUF, group, 0)

        plsc.subcore_barrier()
        pltpu.sync_copy(acc.at[pl.ds(r0, RPT), :],
                        out_hbm.at[cid, pl.ds(r0, RPT), :])

    return k


_deg_kernel = _make_deg_kernel()
_msg16 = _make_msg_kernel(D1)
_msg48 = _make_msg_kernel(D2P)


def _mm1_body(x_ref, w_ref, dp0_ref, dp1_ref, hs_ref, dinv_ref):
    deg = dp0_ref[0, :, 0:1] + dp1_ref[0, :, 0:1] + 1.0
    dinv = lax.rsqrt(deg)
    dinv_ref[...] = dinv
    h = jnp.dot(x_ref[...], w_ref[...], preferred_element_type=jnp.float32)
    hs_ref[...] = h * dinv


def _mm1(x, W1, degp):
    return pl.pallas_call(
        _mm1_body,
        grid=(N // RB,),
        in_specs=[
            pl.BlockSpec((RB, 128), lambda i: (i, 0)),
            pl.BlockSpec((128, D1), lambda i: (0, 0)),
            pl.BlockSpec((1, RB, 16), lambda i: (0, i, 0)),
            pl.BlockSpec((1, RB, 16), lambda i: (1, i, 0)),
        ],
        out_specs=[
            pl.BlockSpec((RB, D1), lambda i: (i, 0)),
            pl.BlockSpec((RB, 1), lambda i: (i, 0)),
        ],
        out_shape=[
            jax.ShapeDtypeStruct((N, D1), jnp.float32),
            jax.ShapeDtypeStruct((N, 1), jnp.float32),
        ],
    )(x, W1, degp, degp)


def _comb1_body(a0_ref, a1_ref, hs_ref, dinv_ref, b1_ref, w2_ref, hs2_ref):
    dinv = dinv_ref[...]
    s = (a0_ref[0] + a1_ref[0] + hs_ref[...]) * dinv + b1_ref[...]
    o1 = jnp.maximum(s, 0.0)
    hs2_ref[...] = jnp.dot(
        o1, w2_ref[...], preferred_element_type=jnp.float32) * dinv


def _comb1(acc1, hs1, dinv, b1r, W2):
    return pl.pallas_call(
        _comb1_body,
        grid=(N // RB,),
        in_specs=[
            pl.BlockSpec((1, RB, D1), lambda i: (0, i, 0)),
            pl.BlockSpec((1, RB, D1), lambda i: (1, i, 0)),
            pl.BlockSpec((RB, D1), lambda i: (i, 0)),
            pl.BlockSpec((RB, 1), lambda i: (i, 0)),
            pl.BlockSpec((1, D1), lambda i: (0, 0)),
            pl.BlockSpec((D1, D2P), lambda i: (0, 0)),
        ],
        out_specs=pl.BlockSpec((RB, D2P), lambda i: (i, 0)),
        out_shape=jax.ShapeDtypeStruct((N, D2P), jnp.float32),
    )(acc1, acc1, hs1, dinv, b1r, W2)


def _final_body(a0_ref, a1_ref, hs_ref, dinv_ref, b2_ref, out_ref):
    z = (a0_ref[0] + a1_ref[0] + hs_ref[...]) * dinv_ref[...] + b2_ref[...]
    m = jnp.max(z, axis=1, keepdims=True)
    lse = jnp.log(jnp.sum(jnp.exp(z - m), axis=1, keepdims=True))
    out_ref[...] = z - m - lse


def _final(acc2, hs2, dinv, b2r):
    return pl.pallas_call(
        _final_body,
        grid=(N // RB,),
        in_specs=[
            pl.BlockSpec((1, RB, D2P), lambda i: (0, i, 0)),
            pl.BlockSpec((1, RB, D2P), lambda i: (1, i, 0)),
            pl.BlockSpec((RB, D2P), lambda i: (i, 0)),
            pl.BlockSpec((RB, 1), lambda i: (i, 0)),
            pl.BlockSpec((1, D2P), lambda i: (0, 0)),
        ],
        out_specs=pl.BlockSpec((RB, D2P), lambda i: (i, 0)),
        out_shape=jax.ShapeDtypeStruct((N, D2P), jnp.float32),
    )(acc2, acc2, hs2, dinv, b2r)


def kernel(x, edge_index, W1, b1, W2, b2):
    er = jnp.asarray(edge_index, jnp.int32).reshape(2, NW, CHUNKS, EC)
    b1r = b1.reshape(1, D1)
    b2r = b2.reshape(1, D2P)
    z16 = jnp.zeros((RPT, D1), jnp.float32)
    z40 = jnp.zeros((RPT, D2P), jnp.float32)

    degp = _deg_kernel(er)
    hs1, dinv = _mm1(x, W1, degp)
    acc1 = _msg16(hs1, er, z16)
    hs2 = _comb1(acc1, hs1, dinv, b1r, W2)
    acc2 = _msg48(hs2, er, z40)
    return _final(acc2, hs2, dinv, b2r)
